# baseline (speedup 1.0000x reference)
import functools

import jax
import jax.numpy as jnp
from jax import lax
from jax.experimental import pallas as pl
from jax.experimental.pallas import tpu as pltpu
from jax.experimental.pallas import tpu_sc as plsc

_B = 4096
_K = 26
_V = 1000
_D = 64
_N = _B * _K
_NW = 32
_RPW = _N // _NW
_G = 128
_NG = _RPW // _G
_L = 16


def _sc_gather(w_flat, kjt3):
    mesh = plsc.VectorSubcoreMesh(
        core_axis_name="c", subcore_axis_name="s", num_cores=2, num_subcores=16
    )

    @functools.partial(
        pl.kernel,
        out_type=jax.ShapeDtypeStruct((_N, _D), jnp.float32),
        mesh=mesh,
        scratch_types=[
            pltpu.VMEM((_RPW,), jnp.int32),
            pltpu.VMEM((_NG, _G), jnp.int32),
            pltpu.VMEM((2, _G, _D), jnp.float32),
            pltpu.SemaphoreType.DMA,
            pltpu.SemaphoreType.DMA,
            pltpu.SemaphoreType.DMA,
            pltpu.SemaphoreType.DMA,
        ],
        compiler_params=pltpu.CompilerParams(use_tc_tiling_on_sc=False),
    )
    def k(w_hbm, kjt_hbm, out_hbm, kjt_v, idx_v, rows_v,
          gsem0, gsem1, wsem0, wsem1):
        nc = 2
        wid = lax.axis_index("s") * nc + lax.axis_index("c")
        base = wid * _RPW
        gsems = (gsem0, gsem1)
        wsems = (wsem0, wsem1)

        pltpu.sync_copy(kjt_hbm.at[pl.ds(base, _RPW)], kjt_v)

        def compute_row(r):
            def body(j, _):
                t0 = r * _G + j * _L
                tvec = lax.iota(jnp.int32, _L) + t0
                vals = jnp.clip(kjt_v[pl.ds(t0, _L)], 0, _V - 1)
                idx_v[r, pl.ds(j * _L, _L)] = lax.rem(tvec, _K) * _V + vals
                return 0

            lax.fori_loop(0, _G // _L, body, 0)

        def fire_gather(g, buf):
            return pltpu.async_copy(
                w_hbm.at[idx_v.at[g]], rows_v.at[buf], gsems[buf]
            )

        gc = [None, None]
        wc = [None, None]
        compute_row(0)
        gc[0] = fire_gather(0, 0)
        for g in range(_NG):
            b = g % 2
            nb = 1 - b
            if g + 1 < _NG:
                compute_row(g + 1)
                if wc[nb] is not None:
                    wc[nb].wait()
                gc[nb] = fire_gather(g + 1, nb)
            gc[b].wait()
            wc[b] = pltpu.async_copy(
                rows_v.at[b], out_hbm.at[pl.ds(base + g * _G, _G)], wsems[b]
            )
        wc[0].wait()
        wc[1].wait()

    return k(w_flat, kjt3)


def kernel(kjt, W):
    w_flat = W.reshape(_K * _V, _D)
    out = _sc_gather(w_flat, kjt.astype(jnp.int32))
    return out


# speedup vs baseline: 1.4845x; 1.4845x over previous
import functools

import jax
import jax.numpy as jnp
from jax import lax
from jax.experimental import pallas as pl
from jax.experimental.pallas import tpu as pltpu
from jax.experimental.pallas import tpu_sc as plsc

_B = 4096
_K = 26
_V = 1000
_D = 64
_N = _B * _K
_NW = 32
_RPW = _N // _NW
_G = 128
_NG = _RPW // _G
_L = 16
_NBUF = 4


def _sc_gather(w_flat, kjt):
    mesh = plsc.VectorSubcoreMesh(
        core_axis_name="c", subcore_axis_name="s", num_cores=2, num_subcores=16
    )

    @functools.partial(
        pl.kernel,
        out_type=jax.ShapeDtypeStruct((_N, _D), jnp.float32),
        mesh=mesh,
        scratch_types=[
            pltpu.VMEM((_RPW,), jnp.int32),
            pltpu.VMEM((_NG, _G), jnp.int32),
            pltpu.VMEM((_NBUF, _G, _D), jnp.float32),
            pltpu.SemaphoreType.DMA,
            pltpu.SemaphoreType.DMA,
            pltpu.SemaphoreType.DMA,
            pltpu.SemaphoreType.DMA,
            pltpu.SemaphoreType.DMA,
            pltpu.SemaphoreType.DMA,
            pltpu.SemaphoreType.DMA,
            pltpu.SemaphoreType.DMA,
        ],
        compiler_params=pltpu.CompilerParams(use_tc_tiling_on_sc=False),
    )
    def k(w_hbm, kjt_hbm, out_hbm, kjt_v, idx_v, rows_v,
          gsem0, gsem1, gsem2, gsem3, wsem0, wsem1, wsem2, wsem3):
        nc = 2
        wid = lax.axis_index("s") * nc + lax.axis_index("c")
        base = wid * _RPW
        gsems = (gsem0, gsem1, gsem2, gsem3)
        wsems = (wsem0, wsem1, wsem2, wsem3)

        pltpu.sync_copy(kjt_hbm.at[pl.ds(base, _RPW)], kjt_v)

        def compute_row(r):
            def body(j, _):
                t0 = r * _G + j * _L
                tvec = lax.iota(jnp.int32, _L) + t0
                vals = jnp.clip(kjt_v[pl.ds(t0, _L)], 0, _V - 1)
                idx_v[r, pl.ds(lax.rem(t0, _G), _L)] = (
                    lax.rem(tvec, _K) * _V + vals
                )
                return 0

            lax.fori_loop(0, _G // _L, body, 0)

        def fire_gather(g, buf):
            return pltpu.async_copy(
                w_hbm.at[idx_v.at[g]], rows_v.at[buf], gsems[buf]
            )

        gc = [None] * _NBUF
        wc = [None] * _NBUF
        for g in range(_NBUF - 1):
            compute_row(g)
            gc[g] = fire_gather(g, g)
        for g in range(_NG):
            s = g % _NBUF
            gnext = g + _NBUF - 1
            if gnext < _NG:
                sn = gnext % _NBUF
                compute_row(gnext)
                if wc[sn] is not None:
                    wc[sn].wait()
                gc[sn] = fire_gather(gnext, sn)
            gc[s].wait()
            wc[s] = pltpu.async_copy(
                rows_v.at[s], out_hbm.at[pl.ds(base + g * _G, _G)], wsems[s]
            )
        for s in range(_NBUF):
            if wc[s] is not None:
                wc[s].wait()

    return k(w_flat, kjt)


def kernel(kjt, W):
    w_flat = W.reshape(_K * _V, _D)
    out = _sc_gather(w_flat, kjt.astype(jnp.int32))
    return out.reshape(_B, _K * _D)


# speedup vs baseline: 1.5006x; 1.0109x over previous
import functools

import jax
import jax.numpy as jnp
from jax import lax
from jax.experimental import pallas as pl
from jax.experimental.pallas import tpu as pltpu
from jax.experimental.pallas import tpu_sc as plsc

_B = 4096
_K = 26
_V = 1000
_D = 64
_N = _B * _K
_NW = 32
_RPW = _N // _NW
_G = 128
_NG = _RPW // _G
_L = 16
_NBUF = 8


def _sc_gather(w_flat, kjt):
    mesh = plsc.VectorSubcoreMesh(
        core_axis_name="c", subcore_axis_name="s", num_cores=2, num_subcores=16
    )

    @functools.partial(
        pl.kernel,
        out_type=jax.ShapeDtypeStruct((_N, _D), jnp.float32),
        mesh=mesh,
        scratch_types=[
            pltpu.VMEM((_RPW,), jnp.int32),
            pltpu.VMEM((_NG, _G), jnp.int32),
            pltpu.VMEM((_NBUF, _G, _D), jnp.float32),
        ] + [pltpu.SemaphoreType.DMA] * (2 * _NBUF),
        compiler_params=pltpu.CompilerParams(use_tc_tiling_on_sc=False),
    )
    def k(w_hbm, kjt_hbm, out_hbm, kjt_v, idx_v, rows_v, *sems):
        nc = 2
        wid = lax.axis_index("s") * nc + lax.axis_index("c")
        base = wid * _RPW
        gsems = sems[:_NBUF]
        wsems = sems[_NBUF:]

        pltpu.sync_copy(kjt_hbm.at[pl.ds(base, _RPW)], kjt_v)

        def compute_row(r):
            def body(j, _):
                t0 = r * _G + j * _L
                tvec = lax.iota(jnp.int32, _L) + t0
                vals = jnp.clip(kjt_v[pl.ds(t0, _L)], 0, _V - 1)
                idx_v[r, pl.ds(lax.rem(t0, _G), _L)] = (
                    lax.rem(tvec, _K) * _V + vals
                )
                return 0

            lax.fori_loop(0, _G // _L, body, 0)

        def fire_gather(g, buf):
            return pltpu.async_copy(
                w_hbm.at[idx_v.at[g]], rows_v.at[buf], gsems[buf]
            )

        gc = [None] * _NBUF
        wc = [None] * _NBUF
        for g in range(_NBUF - 1):
            compute_row(g)
            gc[g] = fire_gather(g, g)
        for g in range(_NG):
            s = g % _NBUF
            gnext = g + _NBUF - 1
            if gnext < _NG:
                sn = gnext % _NBUF
                compute_row(gnext)
                if wc[sn] is not None:
                    wc[sn].wait()
                gc[sn] = fire_gather(gnext, sn)
            gc[s].wait()
            wc[s] = pltpu.async_copy(
                rows_v.at[s], out_hbm.at[pl.ds(base + g * _G, _G)], wsems[s]
            )
        for s in range(_NBUF):
            if wc[s] is not None:
                wc[s].wait()

    return k(w_flat, kjt)


def kernel(kjt, W):
    w_flat = W.reshape(_K * _V, _D)
    out = _sc_gather(w_flat, kjt.astype(jnp.int32))
    return out.reshape(_B, _K * _D)


# speedup vs baseline: 1.5013x; 1.0004x over previous
import functools

import jax
import jax.numpy as jnp
from jax import lax
from jax.experimental import pallas as pl
from jax.experimental.pallas import tpu as pltpu
from jax.experimental.pallas import tpu_sc as plsc

_B = 4096
_K = 26
_V = 1000
_D = 64
_N = _B * _K
_NW = 32
_RPW = _N // _NW
_G = 128
_NG = _RPW // _G
_L = 16
_NBUF = 13


def _sc_gather(w_flat, kjt):
    mesh = plsc.VectorSubcoreMesh(
        core_axis_name="c", subcore_axis_name="s", num_cores=2, num_subcores=16
    )

    @functools.partial(
        pl.kernel,
        out_type=jax.ShapeDtypeStruct((_N, _D), jnp.float32),
        mesh=mesh,
        scratch_types=[
            pltpu.VMEM((_RPW,), jnp.int32),
            pltpu.VMEM((_NG, _G), jnp.int32),
            pltpu.VMEM((_NBUF, _G, _D), jnp.float32),
        ] + [pltpu.SemaphoreType.DMA] * (2 * _NBUF),
        compiler_params=pltpu.CompilerParams(use_tc_tiling_on_sc=False),
    )
    def k(w_hbm, kjt_hbm, out_hbm, kjt_v, idx_v, rows_v, *sems):
        nc = 2
        wid = lax.axis_index("s") * nc + lax.axis_index("c")
        base = wid * _RPW
        gsems = sems[:_NBUF]
        wsems = sems[_NBUF:]

        pltpu.sync_copy(kjt_hbm.at[pl.ds(base, _RPW)], kjt_v)

        def compute_row(r):
            def body(j, _):
                t0 = r * _G + j * _L
                tvec = lax.iota(jnp.int32, _L) + t0
                vals = jnp.clip(kjt_v[pl.ds(t0, _L)], 0, _V - 1)
                idx_v[r, pl.ds(lax.rem(t0, _G), _L)] = (
                    lax.rem(tvec, _K) * _V + vals
                )
                return 0

            lax.fori_loop(0, _G // _L, body, 0)

        def fire_gather(g, buf):
            return pltpu.async_copy(
                w_hbm.at[idx_v.at[g]], rows_v.at[buf], gsems[buf]
            )

        gc = [None] * _NBUF
        wc = [None] * _NBUF
        for g in range(_NBUF - 1):
            compute_row(g)
            gc[g] = fire_gather(g, g)
        for g in range(_NG):
            s = g % _NBUF
            gnext = g + _NBUF - 1
            if gnext < _NG:
                sn = gnext % _NBUF
                compute_row(gnext)
                if wc[sn] is not None:
                    wc[sn].wait()
                gc[sn] = fire_gather(gnext, sn)
            gc[s].wait()
            wc[s] = pltpu.async_copy(
                rows_v.at[s], out_hbm.at[pl.ds(base + g * _G, _G)], wsems[s]
            )
        for s in range(_NBUF):
            if wc[s] is not None:
                wc[s].wait()

    return k(w_flat, kjt)


def kernel(kjt, W):
    w_flat = W.reshape(_K * _V, _D)
    out = _sc_gather(w_flat, kjt.astype(jnp.int32))
    return out.reshape(_B, _K * _D)
